# baseline (speedup 1.0000x reference)
import functools

import jax
import jax.numpy as jnp
from jax import lax
from jax.experimental import pallas as pl
from jax.experimental.pallas import tpu as pltpu, tpu_sc as plsc

_INFO = plsc.get_sparse_core_info()
_NS, _L = _INFO.num_subcores, _INFO.num_lanes

_B, _S = 16, 2048
_NQ = 4
_Q = _S // _NQ


def _make_sc_call():
    mesh = plsc.VectorSubcoreMesh(core_axis_name="c", subcore_axis_name="s",
                                  num_cores=1)

    @functools.partial(
        pl.kernel,
        mesh=mesh,
        out_type=jax.ShapeDtypeStruct((_B, _S), jnp.float32),
        scratch_types=[
            pltpu.VMEM((_S,), jnp.float32),
            pltpu.VMEM((_S,), jnp.int32),
            pltpu.VMEM((_L,), jnp.float32),
            pltpu.SemaphoreType.DMA,
            pltpu.SemaphoreType.DMA,
        ],
    )
    def sc_kernel(scores_hbm, mask_hbm, bias_hbm, out_hbm,
                  scores_v, mask_v, bias_v, sem, out_sem):
        row = lax.axis_index("s") + lax.axis_index("c")
        db = pltpu.async_copy(bias_hbm, bias_v.at[pl.ds(0, 1)], sem)
        ins = []
        for q in range(_NQ):
            qs = pl.ds(q * _Q, _Q)
            ins.append((
                pltpu.async_copy(scores_hbm.at[row, qs], scores_v.at[qs], sem),
                pltpu.async_copy(mask_hbm.at[row, qs], mask_v.at[qs], sem),
            ))
        db.wait()
        bias_vec = jnp.full((_L,), bias_v[...][0], dtype=jnp.float32)
        outs = []
        for q in range(_NQ):
            dq_s, dq_m = ins[q]
            dq_s.wait()
            dq_m.wait()
            for i in range(q * _Q // _L, (q + 1) * _Q // _L):
                sl = pl.ds(i * _L, _L)
                scores_v[sl] = ((scores_v[sl] + bias_vec)
                                * mask_v[sl].astype(jnp.float32))
            qs = pl.ds(q * _Q, _Q)
            outs.append(
                pltpu.async_copy(scores_v.at[qs], out_hbm.at[row, qs], out_sem))
        for w in outs:
            w.wait()

    return sc_kernel


_SC_CALL = _make_sc_call()


@jax.jit
def kernel(sent_group_scores, sel_sent_emb, sel_sent_masks, group_embs,
           candi_sent_masks, bias):
    return _SC_CALL(sent_group_scores, candi_sent_masks,
                    jnp.reshape(bias, (1,)))
